# baseline (speedup 1.0000x reference)
import functools

import jax
import jax.numpy as jnp
from jax import lax
from jax.experimental import pallas as pl
from jax.experimental.pallas import tpu as pltpu
from jax.experimental.pallas import tpu_sc as plsc

N = 10000
E = 320000
D = 128
H = 4
DH = 32
HD = H * DH
NC = 2
NS = 16
NW = NC * NS
EPW = E // NW
CH = 40
NCH = EPW // CH
IB = 10
NB = NCH // IB
NP = 10240
NPD = NP // 8
TN = NP // NS
TND = NPD // NS
DW = 16


def _proj_body(x_ref, ws_ref, wr_ref, bs_ref, br_ref, xs_ref, xr_ref):
    xb = x_ref[...]
    xs_ref[...] = jnp.dot(xb, ws_ref[...], preferred_element_type=jnp.float32) + bs_ref[...]
    xr_ref[...] = jnp.dot(xb, wr_ref[...], preferred_element_type=jnp.float32) + br_ref[...]


def _project(x, Ws, Wr, bs, br):
    BR = 1000
    return pl.pallas_call(
        _proj_body,
        grid=(N // BR,),
        in_specs=[
            pl.BlockSpec((BR, D), lambda i: (i, 0)),
            pl.BlockSpec((D, HD), lambda i: (0, 0)),
            pl.BlockSpec((D, HD), lambda i: (0, 0)),
            pl.BlockSpec((1, HD), lambda i: (0, 0)),
            pl.BlockSpec((1, HD), lambda i: (0, 0)),
        ],
        out_specs=[
            pl.BlockSpec((BR, HD), lambda i: (i, 0)),
            pl.BlockSpec((BR, HD), lambda i: (i, 0)),
        ],
        out_shape=[
            jax.ShapeDtypeStruct((N, HD), jnp.float32),
            jax.ShapeDtypeStruct((N, HD), jnp.float32),
        ],
    )(x, Ws, Wr, bs, br)


@functools.cache
def _edge_pass_kernel():
    return pl.kernel(
        _edge_body,
        out_type=[
            jax.ShapeDtypeStruct((2, NP, HD), jnp.float32),
            jax.ShapeDtypeStruct((2, NPD, HD), jnp.float32),
        ],
        mesh=plsc.VectorSubcoreMesh(
            core_axis_name="c", subcore_axis_name="s", num_cores=NC, num_subcores=NS
        ),
        scratch_types=[
            pltpu.VMEM((IB * CH,), jnp.int32),
            pltpu.VMEM((IB * CH,), jnp.int32),
            pltpu.VMEM((CH,), jnp.int32),
            pltpu.VMEM((CH,), jnp.int32),
            pltpu.VMEM((CH, HD), jnp.float32),
            pltpu.VMEM((CH, HD), jnp.float32),
            pltpu.VMEM((CH, HD), jnp.float32),
            pltpu.VMEM((CH, HD), jnp.float32),
            pltpu.VMEM((CH, HD), jnp.float32),
            pltpu.VMEM((CH, HD), jnp.float32),
            pltpu.VMEM((DH,), jnp.float32),
            pltpu.VMEM_SHARED((NP, HD), jnp.float32),
            pltpu.VMEM_SHARED((NPD, HD), jnp.float32),
            pltpu.SemaphoreType.DMA,
            pltpu.SemaphoreType.DMA,
        ],
    )


def _edge_body(send_ref, recv_ref, xs_ref, xr_ref, a_ref, z128_ref,
               num_out, den_out,
               sbig, rbig, ridx_s, ridx2, srow_e, rrow_e, srow_o, rrow_o,
               msg, den, a_v, acc_num, acc_den, sem, sem2):
    c = lax.axis_index("c")
    s = lax.axis_index("s")
    wid = s * NC + c

    r0 = s * TN
    r0d = s * TND
    pltpu.sync_copy(z128_ref.at[pl.ds(r0, TN)], acc_num.at[pl.ds(r0, TN)])
    pltpu.sync_copy(z128_ref.at[pl.ds(0, TND)], acc_den.at[pl.ds(r0d, TND)])
    pltpu.sync_copy(a_ref, a_v)
    plsc.subcore_barrier()

    a0 = a_v[pl.ds(0, 16)]
    a1 = a_v[pl.ds(16, 16)]
    lane = lax.iota(jnp.int32, 16)
    onehot = [jnp.where(lane == h, 1.0, 0.0).astype(jnp.float32) for h in range(H)]
    perms = [lane ^ (1 << k) for k in range(4)]
    base0 = wid * EPW

    dnums = lax.GatherDimensionNumbers(
        offset_dims=(), collapsed_slice_dims=(0,), start_index_map=(0,)
    )

    def _shuf(v, p):
        return lax.gather(
            v, p[:, None], dimension_numbers=dnums, slice_sizes=(1,),
            mode=lax.GatherScatterMode.PROMISE_IN_BOUNDS,
        )

    def _hsum(v):
        for p in perms:
            v = v + _shuf(v, p)
        return v

    wins = sorted({min(16 * k, CH - 16) for k in range((CH + 15) // 16)})
    rows = [(srow_e, rrow_e), (srow_o, rrow_o)]

    def _issue_gather(j):
        sr, rr = rows[j % 2]
        d1 = pltpu.async_copy(xs_ref.at[sbig.at[pl.ds(CH * j, CH)]], sr, sem)
        d2 = pltpu.async_copy(xr_ref.at[rbig.at[pl.ds(CH * j, CH)]], rr, sem)
        return d1, d2

    def _compute(j):
        srow, rrow = rows[j % 2]
        jb = CH * j
        mlo = jnp.where(lane < 8, 1.0, 0.0).astype(jnp.float32)
        mhi = 1.0 - mlo
        p_swap8 = lane ^ 8
        p_lo = lane & 7
        p_hi = (lane & 7) | 8
        p01 = jnp.where(lane == 1, 8, 0)
        p23 = jnp.where(lane == 3, 8, 0)
        m01 = jnp.where(lane < 2, 1.0, 0.0).astype(jnp.float32)
        m23 = (jnp.where(lane < 4, 1.0, 0.0) - jnp.where(lane < 2, 1.0, 0.0)).astype(jnp.float32)

        def do_edge(i):
            sl = [srow[i, pl.ds(16 * jj, 16)] for jj in range(8)]
            ml = []
            for jj in range(8):
                z = sl[jj] + rrow[i, pl.ds(16 * jj, 16)]
                t = jnp.exp(z)
                w = t * (t + 2.0)
                m = z * (1.0 - 2.0 / (w + 2.0))
                ml.append(m * (a0 if jj % 2 == 0 else a1))
            evs = []
            for hp in range(2):
                accA = ml[4 * hp] + ml[4 * hp + 1]
                accB = ml[4 * hp + 2] + ml[4 * hp + 3]
                v = accA * mlo + _shuf(accB, p_swap8) * mhi
                for st in range(3):
                    v = v + _shuf(v, lane ^ (1 << st))
                ev = jnp.exp(v)
                eA = _shuf(ev, p_lo)
                eB = _shuf(ev, p_hi)
                msg[i, pl.ds(64 * hp, 16)] = sl[4 * hp] * eA
                msg[i, pl.ds(64 * hp + 16, 16)] = sl[4 * hp + 1] * eA
                msg[i, pl.ds(64 * hp + 32, 16)] = sl[4 * hp + 2] * eB
                msg[i, pl.ds(64 * hp + 48, 16)] = sl[4 * hp + 3] * eB
                evs.append(ev)
            dvec = _shuf(evs[0], p01) * m01 + _shuf(evs[1], p23) * m23
            w0 = lax.min((i >> 4) << 4, CH - 16)
            rv = rbig[pl.ds(jb + w0, 16)] & 7
            slotf = _shuf(rv, jnp.full((16,), i - w0, jnp.int32)).astype(jnp.float32)
            for k in range(8):
                dk = slotf - float(k)
                mk = jnp.maximum(1.0 - dk * dk, 0.0)
                den[i, pl.ds(16 * k, 16)] = dvec * mk

        def edge2(i2, carry2):
            do_edge(2 * i2)
            do_edge(2 * i2 + 1)
            return carry2

        lax.fori_loop(0, CH // 2, edge2, 0)

    def block(b, carry):
        base = base0 + b * (IB * CH)
        pltpu.sync_copy(send_ref.at[pl.ds(base, IB * CH)], sbig)
        pltpu.sync_copy(recv_ref.at[pl.ds(base, IB * CH)], rbig)
        pend = {0: _issue_gather(0), 1: _issue_gather(1)}
        for j in range(IB):
            d1, d2 = pend.pop(j)
            d1.wait()
            d2.wait()
            _compute(j)
            for w in wins:
                rv = rbig[pl.ds(CH * j + w, 16)]
                ridx_s[pl.ds(w, 16)] = rv
                ridx2[pl.ds(w, 16)] = lax.shift_right_logical(rv, 3)
            s1 = pltpu.async_copy(msg, acc_num.at[ridx_s], sem2, add=True)
            s2 = pltpu.async_copy(den, acc_den.at[ridx2], sem2, add=True)
            if j + 2 < IB:
                pend[j + 2] = _issue_gather(j + 2)
            s1.wait()
            s2.wait()
        return carry

    lax.fori_loop(0, NB, block, 0)
    plsc.subcore_barrier()

    pltpu.sync_copy(acc_num.at[pl.ds(r0, TN)], num_out.at[c, pl.ds(r0, TN)])
    pltpu.sync_copy(acc_den.at[pl.ds(r0d, TND)], den_out.at[c, pl.ds(r0d, TND)])


def _combine_body(n0_ref, n1_ref, d0_ref, d1_ref, o_ref):
    den16 = d0_ref[0] + d1_ref[0]
    row = lax.broadcasted_iota(jnp.int32, (DW, HD), 0)
    col = lax.broadcasted_iota(jnp.int32, (DW, HD), 1)
    expand = jnp.where(row == col // DH, 1.0, 0.0).astype(jnp.float32)
    den = jnp.dot(den16, expand, preferred_element_type=jnp.float32)
    num = n0_ref[0] + n1_ref[0]
    o_ref[...] = num / jnp.maximum(den, 1e-30)


def _combine(num_p, den_p):
    BR = 1000
    return pl.pallas_call(
        _combine_body,
        grid=(N // BR,),
        in_specs=[
            pl.BlockSpec((1, BR, HD), lambda i: (0, i, 0)),
            pl.BlockSpec((1, BR, HD), lambda i: (1, i, 0)),
            pl.BlockSpec((1, BR, DW), lambda i: (0, i, 0)),
            pl.BlockSpec((1, BR, DW), lambda i: (1, i, 0)),
        ],
        out_specs=pl.BlockSpec((BR, HD), lambda i: (i, 0)),
        out_shape=jax.ShapeDtypeStruct((N, HD), jnp.float32),
    )(num_p, num_p, den_p, den_p)


def kernel(x, edge_index, Ws_k, Ws_b, Wr_k, Wr_b, A_k, A_b):
    del A_b
    senders = edge_index[0].astype(jnp.int32)
    receivers = edge_index[1].astype(jnp.int32)
    Ws = Ws_k.reshape(D, HD)
    Wr = Wr_k.reshape(D, HD)
    bs = Ws_b.reshape(1, HD)
    br = Wr_b.reshape(1, HD)
    a = A_k.reshape(DH)
    xs, xr = _project(x, Ws, Wr, bs, br)
    z128 = jnp.zeros((NP, HD), jnp.float32)
    num_p, den_p = _edge_pass_kernel()(senders, receivers, xs, xr, a, z128)
    den_u = den_p.reshape(2, NP, DW)
    return _combine(num_p, den_u)


# speedup vs baseline: 1.6612x; 1.6612x over previous
import functools

import jax
import jax.numpy as jnp
from jax import lax
from jax.experimental import pallas as pl
from jax.experimental.pallas import tpu as pltpu
from jax.experimental.pallas import tpu_sc as plsc

N = 10000
E = 320000
D = 128
H = 4
DH = 32
HD = H * DH
NC = 2
NS = 16
NW = NC * NS
EPW = E // NW
CH = 40
NCH = EPW // CH
IB = 10
NB = NCH // IB
NP = 10240
NPD = NP // 8
TN = NP // NS
TND = NPD // NS
DW = 16


def _proj_body(x_ref, ws_ref, wr_ref, bs_ref, br_ref, xs_ref, xr_ref):
    xb = x_ref[...]
    xs_ref[...] = jnp.dot(xb, ws_ref[...], preferred_element_type=jnp.float32) + bs_ref[...]
    xr_ref[...] = jnp.dot(xb, wr_ref[...], preferred_element_type=jnp.float32) + br_ref[...]


def _project(x, Ws, Wr, bs, br):
    BR = 1000
    return pl.pallas_call(
        _proj_body,
        grid=(N // BR,),
        in_specs=[
            pl.BlockSpec((BR, D), lambda i: (i, 0)),
            pl.BlockSpec((D, HD), lambda i: (0, 0)),
            pl.BlockSpec((D, HD), lambda i: (0, 0)),
            pl.BlockSpec((1, HD), lambda i: (0, 0)),
            pl.BlockSpec((1, HD), lambda i: (0, 0)),
        ],
        out_specs=[
            pl.BlockSpec((BR, HD), lambda i: (i, 0)),
            pl.BlockSpec((BR, HD), lambda i: (i, 0)),
        ],
        out_shape=[
            jax.ShapeDtypeStruct((N, HD), jnp.float32),
            jax.ShapeDtypeStruct((N, HD), jnp.float32),
        ],
    )(x, Ws, Wr, bs, br)


@functools.cache
def _edge_pass_kernel():
    return pl.kernel(
        _edge_body,
        out_type=[
            jax.ShapeDtypeStruct((2, NP, HD), jnp.float32),
            jax.ShapeDtypeStruct((2, NPD, HD), jnp.float32),
        ],
        mesh=plsc.VectorSubcoreMesh(
            core_axis_name="c", subcore_axis_name="s", num_cores=NC, num_subcores=NS
        ),
        scratch_types=[
            pltpu.VMEM((IB * CH,), jnp.int32),
            pltpu.VMEM((IB * CH,), jnp.int32),
            pltpu.VMEM((CH,), jnp.int32),
            pltpu.VMEM((CH,), jnp.int32),
            pltpu.VMEM((CH, HD), jnp.float32),
            pltpu.VMEM((CH, HD), jnp.float32),
            pltpu.VMEM((CH, HD), jnp.float32),
            pltpu.VMEM((CH, HD), jnp.float32),
            pltpu.VMEM((CH, HD), jnp.float32),
            pltpu.VMEM((CH, HD), jnp.float32),
            pltpu.VMEM((DH,), jnp.float32),
            pltpu.VMEM_SHARED((NP, HD), jnp.float32),
            pltpu.VMEM_SHARED((NPD, HD), jnp.float32),
            pltpu.SemaphoreType.DMA,
            pltpu.SemaphoreType.DMA,
        ],
    )


def _edge_body(send_ref, recv_ref, xs_ref, xr_ref, a_ref, z128_ref,
               num_out, den_out,
               sbig, rbig, ridx_s, ridx2, srow_e, rrow_e, srow_o, rrow_o,
               msg, den, a_v, acc_num, acc_den, sem, sem2):
    c = lax.axis_index("c")
    s = lax.axis_index("s")
    wid = s * NC + c

    r0 = s * TN
    r0d = s * TND
    pltpu.sync_copy(z128_ref.at[pl.ds(r0, TN)], acc_num.at[pl.ds(r0, TN)])
    pltpu.sync_copy(z128_ref.at[pl.ds(0, TND)], acc_den.at[pl.ds(r0d, TND)])
    pltpu.sync_copy(a_ref, a_v)
    plsc.subcore_barrier()

    a0 = a_v[pl.ds(0, 16)]
    a1 = a_v[pl.ds(16, 16)]
    lane = lax.iota(jnp.int32, 16)
    onehot = [jnp.where(lane == h, 1.0, 0.0).astype(jnp.float32) for h in range(H)]
    perms = [lane ^ (1 << k) for k in range(4)]
    base0 = wid * EPW

    dnums = lax.GatherDimensionNumbers(
        offset_dims=(), collapsed_slice_dims=(0,), start_index_map=(0,)
    )

    def _shuf(v, p):
        return lax.gather(
            v, p[:, None], dimension_numbers=dnums, slice_sizes=(1,),
            mode=lax.GatherScatterMode.PROMISE_IN_BOUNDS,
        )

    def _hsum(v):
        for p in perms:
            v = v + _shuf(v, p)
        return v

    wins = sorted({min(16 * k, CH - 16) for k in range((CH + 15) // 16)})
    rows = [(srow_e, rrow_e), (srow_o, rrow_o)]

    def _issue_gather(j):
        sr, rr = rows[j % 2]
        d1 = pltpu.async_copy(xs_ref.at[sbig.at[pl.ds(CH * j, CH)]], sr, sem)
        d2 = pltpu.async_copy(xr_ref.at[rbig.at[pl.ds(CH * j, CH)]], rr, sem)
        return d1, d2

    def _compute(j):
        srow, rrow = rows[j % 2]
        jb = CH * j

        def edge(i, carry2):
            sl = [srow[i, pl.ds(16 * jj, 16)] for jj in range(8)]
            ml = []
            for jj in range(8):
                z = sl[jj] + rrow[i, pl.ds(16 * jj, 16)]
                t = jnp.exp(z)
                w = t * (t + 2.0)
                m = z * (1.0 - 2.0 / (w + 2.0))
                ml.append(m * (a0 if jj % 2 == 0 else a1))
            dvec = jnp.zeros((16,), jnp.float32)
            for h in range(H):
                logit = _hsum(ml[2 * h] + ml[2 * h + 1])
                ev = jnp.exp(logit)
                msg[i, pl.ds(32 * h, 16)] = sl[2 * h] * ev
                msg[i, pl.ds(32 * h + 16, 16)] = sl[2 * h + 1] * ev
                dvec = dvec + ev * onehot[h]
            w0 = lax.min((i >> 4) << 4, CH - 16)
            rv = rbig[pl.ds(jb + w0, 16)] & 7
            slotf = _shuf(rv, jnp.full((16,), i - w0, jnp.int32)).astype(jnp.float32)
            for k in range(8):
                dk = slotf - float(k)
                mk = jnp.maximum(1.0 - dk * dk, 0.0)
                den[i, pl.ds(16 * k, 16)] = dvec * mk
            return carry2

        lax.fori_loop(0, CH, edge, 0)

    def block(b, carry):
        base = base0 + b * (IB * CH)
        pltpu.sync_copy(send_ref.at[pl.ds(base, IB * CH)], sbig)
        pltpu.sync_copy(recv_ref.at[pl.ds(base, IB * CH)], rbig)
        pend = {0: _issue_gather(0), 1: _issue_gather(1)}
        for j in range(IB):
            d1, d2 = pend.pop(j)
            d1.wait()
            d2.wait()
            _compute(j)
            for w in wins:
                rv = rbig[pl.ds(CH * j + w, 16)]
                ridx_s[pl.ds(w, 16)] = rv
                ridx2[pl.ds(w, 16)] = lax.shift_right_logical(rv, 3)
            s1 = pltpu.async_copy(msg, acc_num.at[ridx_s], sem2, add=True)
            s2 = pltpu.async_copy(den, acc_den.at[ridx2], sem2, add=True)
            if j + 2 < IB:
                pend[j + 2] = _issue_gather(j + 2)
            s1.wait()
            s2.wait()
        return carry

    lax.fori_loop(0, NB, block, 0)
    plsc.subcore_barrier()

    pltpu.sync_copy(acc_num.at[pl.ds(r0, TN)], num_out.at[c, pl.ds(r0, TN)])
    pltpu.sync_copy(acc_den.at[pl.ds(r0d, TND)], den_out.at[c, pl.ds(r0d, TND)])


def _combine_body(n0_ref, n1_ref, d0_ref, d1_ref, o_ref):
    den16 = d0_ref[0] + d1_ref[0]
    row = lax.broadcasted_iota(jnp.int32, (DW, HD), 0)
    col = lax.broadcasted_iota(jnp.int32, (DW, HD), 1)
    expand = jnp.where(row == col // DH, 1.0, 0.0).astype(jnp.float32)
    den = jnp.dot(den16, expand, preferred_element_type=jnp.float32)
    num = n0_ref[0] + n1_ref[0]
    o_ref[...] = num / jnp.maximum(den, 1e-30)


def _combine(num_p, den_p):
    BR = 1000
    return pl.pallas_call(
        _combine_body,
        grid=(N // BR,),
        in_specs=[
            pl.BlockSpec((1, BR, HD), lambda i: (0, i, 0)),
            pl.BlockSpec((1, BR, HD), lambda i: (1, i, 0)),
            pl.BlockSpec((1, BR, DW), lambda i: (0, i, 0)),
            pl.BlockSpec((1, BR, DW), lambda i: (1, i, 0)),
        ],
        out_specs=pl.BlockSpec((BR, HD), lambda i: (i, 0)),
        out_shape=jax.ShapeDtypeStruct((N, HD), jnp.float32),
    )(num_p, num_p, den_p, den_p)


def kernel(x, edge_index, Ws_k, Ws_b, Wr_k, Wr_b, A_k, A_b):
    del A_b
    senders = edge_index[0].astype(jnp.int32)
    receivers = edge_index[1].astype(jnp.int32)
    Ws = Ws_k.reshape(D, HD)
    Wr = Wr_k.reshape(D, HD)
    bs = Ws_b.reshape(1, HD)
    br = Wr_b.reshape(1, HD)
    a = A_k.reshape(DH)
    xs, xr = _project(x, Ws, Wr, bs, br)
    z128 = jnp.zeros((NP, HD), jnp.float32)
    num_p, den_p = _edge_pass_kernel()(senders, receivers, xs, xr, a, z128)
    den_u = den_p.reshape(2, NP, DW)
    return _combine(num_p, den_u)


# speedup vs baseline: 1.6817x; 1.0124x over previous
import functools

import jax
import jax.numpy as jnp
from jax import lax
from jax.experimental import pallas as pl
from jax.experimental.pallas import tpu as pltpu
from jax.experimental.pallas import tpu_sc as plsc

N = 10000
E = 320000
D = 128
H = 4
DH = 32
HD = H * DH
NC = 2
NS = 16
NW = NC * NS
EPW = E // NW
CH = 40
NCH = EPW // CH
IB = 10
NB = NCH // IB
NP = 10240
NPD = NP // 8
TN = NP // NS
TND = NPD // NS
DW = 16


def _proj_body(x_ref, ws_ref, wr_ref, bs_ref, br_ref, xs_ref, xr_ref):
    xb = x_ref[...]
    xs_ref[...] = jnp.dot(xb, ws_ref[...], preferred_element_type=jnp.float32) + bs_ref[...]
    xr_ref[...] = jnp.dot(xb, wr_ref[...], preferred_element_type=jnp.float32) + br_ref[...]


def _project(x, Ws, Wr, bs, br):
    BR = 1000
    return pl.pallas_call(
        _proj_body,
        grid=(N // BR,),
        in_specs=[
            pl.BlockSpec((BR, D), lambda i: (i, 0)),
            pl.BlockSpec((D, HD), lambda i: (0, 0)),
            pl.BlockSpec((D, HD), lambda i: (0, 0)),
            pl.BlockSpec((1, HD), lambda i: (0, 0)),
            pl.BlockSpec((1, HD), lambda i: (0, 0)),
        ],
        out_specs=[
            pl.BlockSpec((BR, HD), lambda i: (i, 0)),
            pl.BlockSpec((BR, HD), lambda i: (i, 0)),
        ],
        out_shape=[
            jax.ShapeDtypeStruct((N, HD), jnp.float32),
            jax.ShapeDtypeStruct((N, HD), jnp.float32),
        ],
    )(x, Ws, Wr, bs, br)


@functools.cache
def _edge_pass_kernel():
    return pl.kernel(
        _edge_body,
        out_type=[
            jax.ShapeDtypeStruct((2, NP, HD), jnp.float32),
            jax.ShapeDtypeStruct((2, NPD, HD), jnp.float32),
        ],
        mesh=plsc.VectorSubcoreMesh(
            core_axis_name="c", subcore_axis_name="s", num_cores=NC, num_subcores=NS
        ),
        scratch_types=[
            pltpu.VMEM((IB * CH,), jnp.int32),
            pltpu.VMEM((IB * CH,), jnp.int32),
            pltpu.VMEM((CH,), jnp.int32),
            pltpu.VMEM((CH,), jnp.int32),
            pltpu.VMEM((CH, HD), jnp.float32),
            pltpu.VMEM((CH, HD), jnp.float32),
            pltpu.VMEM((CH, HD), jnp.float32),
            pltpu.VMEM((CH, HD), jnp.float32),
            pltpu.VMEM((CH, HD), jnp.float32),
            pltpu.VMEM((CH, HD), jnp.float32),
            pltpu.VMEM((DH,), jnp.float32),
            pltpu.VMEM_SHARED((NP, HD), jnp.float32),
            pltpu.VMEM_SHARED((NPD, HD), jnp.float32),
            pltpu.SemaphoreType.DMA,
            pltpu.SemaphoreType.DMA,
        ],
    )


def _edge_body(send_ref, recv_ref, xs_ref, xr_ref, a_ref, z128_ref,
               num_out, den_out,
               sbig, rbig, ridx_s, ridx2, srow_e, rrow_e, srow_o, rrow_o,
               msg, den, a_v, acc_num, acc_den, sem, sem2):
    c = lax.axis_index("c")
    s = lax.axis_index("s")
    wid = s * NC + c

    r0 = s * TN
    r0d = s * TND
    pltpu.sync_copy(z128_ref.at[pl.ds(r0, TN)], acc_num.at[pl.ds(r0, TN)])
    pltpu.sync_copy(z128_ref.at[pl.ds(0, TND)], acc_den.at[pl.ds(r0d, TND)])
    pltpu.sync_copy(a_ref, a_v)
    plsc.subcore_barrier()

    a0 = a_v[pl.ds(0, 16)]
    a1 = a_v[pl.ds(16, 16)]
    lane = lax.iota(jnp.int32, 16)
    onehot = [jnp.where(lane == h, 1.0, 0.0).astype(jnp.float32) for h in range(H)]
    perms = [lane ^ (1 << k) for k in range(4)]
    base0 = wid * EPW

    dnums = lax.GatherDimensionNumbers(
        offset_dims=(), collapsed_slice_dims=(0,), start_index_map=(0,)
    )

    def _shuf(v, p):
        return lax.gather(
            v, p[:, None], dimension_numbers=dnums, slice_sizes=(1,),
            mode=lax.GatherScatterMode.PROMISE_IN_BOUNDS,
        )

    def _hsum(v):
        for p in perms:
            v = v + _shuf(v, p)
        return v

    wins = sorted({min(16 * k, CH - 16) for k in range((CH + 15) // 16)})
    rows = [(srow_e, rrow_e), (srow_o, rrow_o)]

    def _issue_gather(j):
        sr, rr = rows[j % 2]
        d1 = pltpu.async_copy(xs_ref.at[sbig.at[pl.ds(CH * j, CH)]], sr, sem)
        d2 = pltpu.async_copy(xr_ref.at[rbig.at[pl.ds(CH * j, CH)]], rr, sem)
        return d1, d2

    def _compute(j):
        srow, rrow = rows[j % 2]
        jb = CH * j

        def edge(i, carry2):
            sl = [srow[i, pl.ds(16 * jj, 16)] for jj in range(8)]
            ml = []
            for jj in range(8):
                z = sl[jj] + rrow[i, pl.ds(16 * jj, 16)]
                t = jnp.exp(z)
                w = t * (t + 2.0)
                m = z * (1.0 - 2.0 / (w + 2.0))
                ml.append(m * (a0 if jj % 2 == 0 else a1))
            dvec = jnp.zeros((16,), jnp.float32)
            for h in range(H):
                logit = _hsum(ml[2 * h] + ml[2 * h + 1])
                ev = jnp.exp(logit)
                msg[i, pl.ds(32 * h, 16)] = sl[2 * h] * ev
                msg[i, pl.ds(32 * h + 16, 16)] = sl[2 * h + 1] * ev
                dvec = dvec + ev * onehot[h]
            w0 = lax.min((i >> 4) << 4, CH - 16)
            rv = rbig[pl.ds(jb + w0, 16)] & 7
            slotf = _shuf(rv, jnp.full((16,), i - w0, jnp.int32)).astype(jnp.float32)
            for k in range(8):
                dk = slotf - float(k)
                mk = jnp.maximum(1.0 - dk * dk, 0.0)
                den[i, pl.ds(16 * k, 16)] = dvec * mk
            return carry2

        lax.fori_loop(0, CH, edge, 0)

    def block(b, carry):
        base = base0 + b * (IB * CH)
        di1 = pltpu.async_copy(send_ref.at[pl.ds(base, IB * CH)], sbig, sem2)
        di2 = pltpu.async_copy(recv_ref.at[pl.ds(base, IB * CH)], rbig, sem2)
        di1.wait()
        di2.wait()
        pend = {0: _issue_gather(0), 1: _issue_gather(1)}
        for j in range(IB):
            d1, d2 = pend.pop(j)
            d1.wait()
            d2.wait()
            _compute(j)
            for w in wins:
                rv = rbig[pl.ds(CH * j + w, 16)]
                ridx_s[pl.ds(w, 16)] = rv
                ridx2[pl.ds(w, 16)] = lax.shift_right_logical(rv, 3)
            s1 = pltpu.async_copy(msg, acc_num.at[ridx_s], sem2, add=True)
            s2 = pltpu.async_copy(den, acc_den.at[ridx2], sem2, add=True)
            if j + 2 < IB:
                pend[j + 2] = _issue_gather(j + 2)
            s1.wait()
            s2.wait()
        return carry

    lax.fori_loop(0, NB, block, 0)
    plsc.subcore_barrier()

    pltpu.sync_copy(acc_num.at[pl.ds(r0, TN)], num_out.at[c, pl.ds(r0, TN)])
    pltpu.sync_copy(acc_den.at[pl.ds(r0d, TND)], den_out.at[c, pl.ds(r0d, TND)])


def _combine_body(n0_ref, n1_ref, d0_ref, d1_ref, o_ref):
    den16 = d0_ref[0] + d1_ref[0]
    row = lax.broadcasted_iota(jnp.int32, (DW, HD), 0)
    col = lax.broadcasted_iota(jnp.int32, (DW, HD), 1)
    expand = jnp.where(row == col // DH, 1.0, 0.0).astype(jnp.float32)
    den = jnp.dot(den16, expand, preferred_element_type=jnp.float32)
    num = n0_ref[0] + n1_ref[0]
    o_ref[...] = num / jnp.maximum(den, 1e-30)


def _combine(num_p, den_p):
    BR = 1000
    return pl.pallas_call(
        _combine_body,
        grid=(N // BR,),
        in_specs=[
            pl.BlockSpec((1, BR, HD), lambda i: (0, i, 0)),
            pl.BlockSpec((1, BR, HD), lambda i: (1, i, 0)),
            pl.BlockSpec((1, BR, DW), lambda i: (0, i, 0)),
            pl.BlockSpec((1, BR, DW), lambda i: (1, i, 0)),
        ],
        out_specs=pl.BlockSpec((BR, HD), lambda i: (i, 0)),
        out_shape=jax.ShapeDtypeStruct((N, HD), jnp.float32),
    )(num_p, num_p, den_p, den_p)


def kernel(x, edge_index, Ws_k, Ws_b, Wr_k, Wr_b, A_k, A_b):
    del A_b
    senders = edge_index[0].astype(jnp.int32)
    receivers = edge_index[1].astype(jnp.int32)
    Ws = Ws_k.reshape(D, HD)
    Wr = Wr_k.reshape(D, HD)
    bs = Ws_b.reshape(1, HD)
    br = Wr_b.reshape(1, HD)
    a = A_k.reshape(DH)
    xs, xr = _project(x, Ws, Wr, bs, br)
    z128 = jnp.zeros((NP, HD), jnp.float32)
    num_p, den_p = _edge_pass_kernel()(senders, receivers, xs, xr, a, z128)
    den_u = den_p.reshape(2, NP, DW)
    return _combine(num_p, den_u)
